# baseline (speedup 1.0000x reference)
import jax
import jax.numpy as jnp
from jax import lax
from jax.experimental import pallas as pl
from jax.experimental.pallas import tpu as pltpu
from jax.experimental.pallas import tpu_sc as plsc

E = 64
N_GROUP = 8
GSZ = E // N_GROUP
D_MODEL = 1024
D_FF = 512
T = 4096
CAP = 256
SCALE = 2.0

BUF_ROWS = E * CAP + CAP
DUMMY = E * CAP

NW = 32
TOK_W = T // NW
CHUNK = 32
NCH = TOK_W // CHUNK

RT = 16
RTOK = T // RT
NEGINF = float("-inf")


def _gate_body(x_ref, gw_ref, b_ref, sb_ref, sc_ref):
    x = x_ref[...]
    lg = lax.dot_general(gw_ref[...], x, (((0,), (1,)), ((), ())))
    sg = 1.0 / (1.0 + jnp.exp(-lg))
    sc_ref[...] = sg
    sb_ref[...] = sg + b_ref[...]


def _gate(x, gate_w, bias):
    TB = 1024
    return pl.pallas_call(
        _gate_body,
        grid=(T // TB,),
        in_specs=[
            pl.BlockSpec((TB, D_MODEL), lambda i: (i, 0)),
            pl.BlockSpec((D_MODEL, E), lambda i: (0, 0)),
            pl.BlockSpec((E, 1), lambda i: (0, 0)),
        ],
        out_specs=[
            pl.BlockSpec((E, TB), lambda i: (0, i)),
            pl.BlockSpec((E, TB), lambda i: (0, i)),
        ],
        out_shape=[
            jax.ShapeDtypeStruct((E, T), jnp.float32),
            jax.ShapeDtypeStruct((E, T), jnp.float32),
        ],
    )(x, gate_w, bias.reshape(E, 1))


def _shared_body(x_ref, su_ref, sd_ref, sh_ref):
    x = x_ref[...]
    h = jnp.dot(x, su_ref[...])
    h = jnp.square(jnp.maximum(h, 0.0))
    sh_ref[...] = jnp.dot(h, sd_ref[...])


def _shared(x, su, sd):
    TB = 512
    return pl.pallas_call(
        _shared_body,
        grid=(T // TB,),
        in_specs=[
            pl.BlockSpec((TB, D_MODEL), lambda i: (i, 0)),
            pl.BlockSpec((D_MODEL, D_MODEL), lambda i: (0, 0)),
            pl.BlockSpec((D_MODEL, D_MODEL), lambda i: (0, 0)),
        ],
        out_specs=pl.BlockSpec((TB, D_MODEL), lambda i: (i, 0)),
        out_shape=jax.ShapeDtypeStruct((T, D_MODEL), jnp.float32),
    )(x, su, sd)


def _route1_body(sb_hbm, sc_hbm,
                 ids_hbm, lp_hbm, w0_hbm, w1_hbm, cnt_hbm,
                 sb_v, sw_v,
                 ids_v, lp_v, w0_v, w1_v, cnt_v):
    s = lax.axis_index("s")
    tok0 = s * RTOK
    lane = lax.iota(jnp.int32, 16)

    pltpu.sync_copy(sb_hbm.at[:, pl.ds(tok0, RTOK)], sb_v)
    pltpu.sync_copy(sc_hbm.at[:, pl.ds(tok0, RTOK)], sw_v)

    def batch_body(j, _):
        sl16 = pl.ds(j * 16, 16)
        gs = []
        for g in range(N_GROUP):
            m1 = sb_v[g * GSZ, sl16]
            m2 = jnp.full((16,), NEGINF, jnp.float32)
            for i in range(1, GSZ):
                sv = sb_v[g * GSZ + i, sl16]
                m2 = jnp.maximum(m2, jnp.minimum(m1, sv))
                m1 = jnp.maximum(m1, sv)
            gs.append(m1 + m2)
        t1 = jnp.full((16,), NEGINF, jnp.float32)
        t2 = t1
        t3 = t1
        t4 = t1
        for g in range(N_GROUP):
            xg = gs[g]
            n1 = jnp.maximum(t1, xg)
            r = jnp.minimum(t1, xg)
            n2 = jnp.maximum(t2, r)
            r = jnp.minimum(t2, r)
            n3 = jnp.maximum(t3, r)
            r = jnp.minimum(t3, r)
            t4 = jnp.maximum(t4, r)
            t1, t2, t3 = n1, n2, n3
        gmask = [gs[g] >= t4 for g in range(N_GROUP)]
        m1 = jnp.full((16,), NEGINF, jnp.float32)
        m2 = m1
        i1 = jnp.zeros((16,), jnp.int32)
        i2 = i1
        u1 = jnp.zeros((16,), jnp.float32)
        u2 = u1
        for e in range(E):
            sv = jnp.where(gmask[e // GSZ], sb_v[e, sl16], NEGINF)
            wv = sw_v[e, sl16]
            ev = jnp.full((16,), e, jnp.int32)
            gt = sv > m1
            cv = jnp.where(gt, m1, sv)
            ci = jnp.where(gt, i1, ev)
            cw = jnp.where(gt, u1, wv)
            m1 = jnp.where(gt, sv, m1)
            i1 = jnp.where(gt, ev, i1)
            u1 = jnp.where(gt, wv, u1)
            gt2 = cv > m2
            m2 = jnp.where(gt2, cv, m2)
            i2 = jnp.where(gt2, ci, i2)
            u2 = jnp.where(gt2, cw, u2)
        inv = SCALE / (u1 + u2)
        fl = lane * 2 + j * 32
        plsc.store_scatter(ids_v, [fl], i1)
        plsc.store_scatter(ids_v, [fl + 1], i2)
        w0_v[sl16] = u1 * inv
        w1_v[sl16] = u2 * inv
        return 0

    lax.fori_loop(0, RTOK // 16, batch_body, 0)

    for c in range(E // 16):
        cnt_v[pl.ds(c * 16, 16)] = jnp.zeros((16,), jnp.int32)

    def pos_body(m, _):
        sl16 = pl.ds(m * 16, 16)
        ev = ids_v[sl16]
        c, last = plsc.scan_count(ev)
        basec = plsc.load_gather(cnt_v, [ev])
        lp_v[sl16] = basec + c - 1
        plsc.store_scatter(cnt_v, [ev], basec + c, mask=last)
        return 0

    lax.fori_loop(0, 2 * RTOK // 16, pos_body, 0)

    asl = pl.ds(2 * tok0, 2 * RTOK)
    tsl = pl.ds(tok0, RTOK)
    pltpu.sync_copy(ids_v, ids_hbm.at[asl])
    pltpu.sync_copy(lp_v, lp_hbm.at[asl])
    pltpu.sync_copy(w0_v, w0_hbm.at[tsl])
    pltpu.sync_copy(w1_v, w1_hbm.at[tsl])
    pltpu.sync_copy(cnt_v, cnt_hbm.at[s])


def _route2_body(ids_hbm, lp_hbm, w0_hbm, w1_hbm, cnt_hbm,
                 sd0_hbm, sd1_hbm, sc0_hbm, sc1_hbm, w0o_hbm, w1o_hbm,
                 ids_v, lp_v, w0_v, w1_v, all_v, base_v,
                 sl_d0, sl_d1, sl_c0, sl_c1):
    s = lax.axis_index("s")
    tok0 = s * RTOK
    lane = lax.iota(jnp.int32, 16)

    asl = pl.ds(2 * tok0, 2 * RTOK)
    tsl = pl.ds(tok0, RTOK)
    pltpu.sync_copy(ids_hbm.at[asl], ids_v)
    pltpu.sync_copy(lp_hbm.at[asl], lp_v)
    pltpu.sync_copy(w0_hbm.at[tsl], w0_v)
    pltpu.sync_copy(w1_hbm.at[tsl], w1_v)
    pltpu.sync_copy(cnt_hbm, all_v)

    for c in range(E // 16):
        acc = jnp.zeros((16,), jnp.int32)
        for r in range(RT):
            pred = jnp.where(jnp.int32(r) < s, 1, 0).astype(jnp.int32)
            acc = acc + all_v[r, pl.ds(c * 16, 16)] * pred
        base_v[pl.ds(c * 16, 16)] = acc

    for j in range(RTOK // 16):
        sl16 = pl.ds(j * 16, 16)
        for k, (w_v, sd_v, sc_slot_v) in enumerate(
            ((w0_v, sl_d0, sl_c0), (w1_v, sl_d1, sl_c1))
        ):
            fl = lane * 2 + (j * 32 + k)
            ev = plsc.load_gather(ids_v, [fl])
            g = plsc.load_gather(base_v, [ev]) + plsc.load_gather(lp_v, [fl])
            valid = g < CAP
            posc = jnp.minimum(g, CAP - 1)
            slotc = ev * CAP + posc
            sd_v[sl16] = jnp.where(valid, slotc, DUMMY)
            sc_slot_v[sl16] = slotc
            w_v[sl16] = jnp.where(valid, w_v[sl16], 0.0)

    pltpu.sync_copy(sl_d0, sd0_hbm.at[tsl])
    pltpu.sync_copy(sl_d1, sd1_hbm.at[tsl])
    pltpu.sync_copy(sl_c0, sc0_hbm.at[tsl])
    pltpu.sync_copy(sl_c1, sc1_hbm.at[tsl])
    pltpu.sync_copy(w0_v, w0o_hbm.at[tsl])
    pltpu.sync_copy(w1_v, w1o_hbm.at[tsl])


def _route(sb, sc):
    mesh = plsc.VectorSubcoreMesh(
        core_axis_name="c", subcore_axis_name="s", num_cores=1)
    i32 = jnp.int32
    f32 = jnp.float32
    cp = pltpu.CompilerParams(needs_layout_passes=False)
    ids, lp, w0, w1, cnts = pl.kernel(
        _route1_body,
        out_type=[
            jax.ShapeDtypeStruct((2 * T,), i32),
            jax.ShapeDtypeStruct((2 * T,), i32),
            jax.ShapeDtypeStruct((T,), f32),
            jax.ShapeDtypeStruct((T,), f32),
            jax.ShapeDtypeStruct((RT, E), i32),
        ],
        mesh=mesh,
        compiler_params=cp,
        scratch_types=[
            pltpu.VMEM((E, RTOK), f32),
            pltpu.VMEM((E, RTOK), f32),
            pltpu.VMEM((2 * RTOK,), i32),
            pltpu.VMEM((2 * RTOK,), i32),
            pltpu.VMEM((RTOK,), f32),
            pltpu.VMEM((RTOK,), f32),
            pltpu.VMEM((E,), i32),
        ],
    )(sb, sc)
    return pl.kernel(
        _route2_body,
        out_type=[jax.ShapeDtypeStruct((T,), i32)] * 4
        + [jax.ShapeDtypeStruct((T,), f32)] * 2,
        mesh=plsc.VectorSubcoreMesh(
            core_axis_name="c", subcore_axis_name="s", num_cores=1),
        compiler_params=cp,
        scratch_types=[
            pltpu.VMEM((2 * RTOK,), i32),
            pltpu.VMEM((2 * RTOK,), i32),
            pltpu.VMEM((RTOK,), f32),
            pltpu.VMEM((RTOK,), f32),
            pltpu.VMEM((RT, E), i32),
            pltpu.VMEM((E,), i32),
            pltpu.VMEM((RTOK,), i32),
            pltpu.VMEM((RTOK,), i32),
            pltpu.VMEM((RTOK,), i32),
            pltpu.VMEM((RTOK,), i32),
        ],
    )(ids, lp, w0, w1, cnts)


def _dispatch_body(x_hbm, sd0_hbm, sd1_hbm, buf_hbm, idx0_v, idx1_v,
                   rows0, rows1, semL0, semL1, semS0, semS1):
    c = lax.axis_index("c")
    s = lax.axis_index("s")
    wid = s * 2 + c
    pltpu.sync_copy(sd0_hbm.at[wid], idx0_v)
    pltpu.sync_copy(sd1_hbm.at[wid], idx1_v)
    rows = (rows0, rows1)
    semL = (semL0, semL1)
    semS = (semS0, semS1)
    lds = {}
    scats = {}
    lds[0] = pltpu.async_copy(
        x_hbm.at[pl.ds(wid * TOK_W, CHUNK)], rows0, semL0)
    for jj in range(NCH):
        b = jj & 1
        if jj + 1 < NCH:
            if jj >= 1:
                scats[jj - 1][0].wait()
                scats[jj - 1][1].wait()
            lds[jj + 1] = pltpu.async_copy(
                x_hbm.at[pl.ds(wid * TOK_W + (jj + 1) * CHUNK, CHUNK)],
                rows[1 - b], semL[1 - b])
        lds[jj].wait()
        scats[jj] = (
            pltpu.async_copy(rows[b], buf_hbm.at[idx0_v.at[jj]], semS[b]),
            pltpu.async_copy(rows[b], buf_hbm.at[idx1_v.at[jj]], semS[b]),
        )
    for jj in (NCH - 2, NCH - 1):
        scats[jj][0].wait()
        scats[jj][1].wait()


def _dispatch(x, sd0, sd1):
    mesh = plsc.VectorSubcoreMesh(core_axis_name="c", subcore_axis_name="s")
    return pl.kernel(
        _dispatch_body,
        out_type=jax.ShapeDtypeStruct((BUF_ROWS, D_MODEL), jnp.float32),
        mesh=mesh,
        scratch_types=[
            pltpu.VMEM((NCH, CHUNK), jnp.int32),
            pltpu.VMEM((NCH, CHUNK), jnp.int32),
            pltpu.VMEM((CHUNK, D_MODEL), jnp.float32),
            pltpu.VMEM((CHUNK, D_MODEL), jnp.float32),
            pltpu.SemaphoreType.DMA,
            pltpu.SemaphoreType.DMA,
            pltpu.SemaphoreType.DMA,
            pltpu.SemaphoreType.DMA,
        ],
    )(x, sd0, sd1)


def _experts_body(buf_ref, up_ref, dn_ref, y_ref):
    h = jnp.dot(buf_ref[...], up_ref[0])
    h = jnp.square(jnp.maximum(h, 0.0))
    y = jnp.dot(h, dn_ref[0])
    yi = pltpu.bitcast(y, jnp.int32) + 0x8000
    yr = yi.reshape(CAP, D_MODEL // 32, 2, 16)
    lo = lax.shift_right_logical(yr[:, :, 0, :], 16)
    hi = jnp.bitwise_and(yr[:, :, 1, :], jnp.int32(-65536))
    y_ref[...] = jnp.bitwise_or(lo, hi).reshape(CAP, D_MODEL // 2)


def _experts(buf, up_w, down_w):
    return pl.pallas_call(
        _experts_body,
        grid=(E,),
        in_specs=[
            pl.BlockSpec((CAP, D_MODEL), lambda i: (i, 0)),
            pl.BlockSpec((1, D_MODEL, D_FF), lambda i: (i, 0, 0)),
            pl.BlockSpec((1, D_FF, D_MODEL), lambda i: (i, 0, 0)),
        ],
        out_specs=pl.BlockSpec((CAP, D_MODEL // 2), lambda i: (i, 0)),
        out_shape=jax.ShapeDtypeStruct((E * CAP, D_MODEL // 2), jnp.int32),
    )(buf, up_w, down_w)


CCH = 16
CNCH = TOK_W // CCH


def _combine_body(y_hbm, sh_hbm, sc0_hbm, sc1_hbm, w0_hbm, w1_hbm, out_hbm,
                  idx0_v, idx1_v, w0_v, w1_v,
                  ya0, yb0, acc0, ya1, yb1, acc1,
                  semG0, semG1, semO0, semO1):
    c = lax.axis_index("c")
    s = lax.axis_index("s")
    wid = s * 2 + c
    pltpu.sync_copy(sc0_hbm.at[wid], idx0_v)
    pltpu.sync_copy(sc1_hbm.at[wid], idx1_v)
    pltpu.sync_copy(w0_hbm.at[wid], w0_v)
    pltpu.sync_copy(w1_hbm.at[wid], w1_v)
    ya = (ya0, ya1)
    yb = (yb0, yb1)
    acc = (acc0, acc1)
    semG = (semG0, semG1)
    semO = (semO0, semO1)

    def issue(jj, b):
        base = wid * TOK_W + jj * CCH
        return (
            pltpu.async_copy(y_hbm.at[idx0_v.at[jj]], ya[b], semG[b]),
            pltpu.async_copy(y_hbm.at[idx1_v.at[jj]], yb[b], semG[b]),
            pltpu.async_copy(sh_hbm.at[pl.ds(base, CCH)], acc[b], semG[b]),
        )

    gaths = {0: issue(0, 0)}
    stores = {}
    for jj in range(CNCH):
        b = jj & 1
        if jj + 1 < CNCH:
            if jj >= 1:
                stores[jj - 1].wait()
            gaths[jj + 1] = issue(jj + 1, 1 - b)
        for d in gaths[jj]:
            d.wait()

        def row_body(t, _):
            fl = jnp.full((16,), jj * CCH + t, jnp.int32)
            w0s = plsc.load_gather(w0_v, [fl])
            w1s = plsc.load_gather(w1_v, [fl])
            av = acc[b]
            yav = ya[b]
            ybv = yb[b]
            for v in range(D_MODEL // 32):
                slp = pl.ds(v * 16, 16)
                alo = pl.ds(v * 32, 16)
                ahi = pl.ds(v * 32 + 16, 16)
                a_lo, a_hi = plsc.unpack(
                    plsc.bitcast(yav[t, slp], jnp.bfloat16),
                    format=plsc.PackFormat.INTERLEAVED)
                b_lo, b_hi = plsc.unpack(
                    plsc.bitcast(ybv[t, slp], jnp.bfloat16),
                    format=plsc.PackFormat.INTERLEAVED)
                av[t, alo] = av[t, alo] + w0s * a_lo + w1s * b_lo
                av[t, ahi] = av[t, ahi] + w0s * a_hi + w1s * b_hi
            return 0

        lax.fori_loop(0, CCH, row_body, 0)
        stores[jj] = pltpu.async_copy(
            acc[b], out_hbm.at[pl.ds(wid * TOK_W + jj * CCH, CCH)], semO[b])
    stores[CNCH - 2].wait()
    stores[CNCH - 1].wait()


def _combine(y, sh, sc0, sc1, w0, w1):
    mesh = plsc.VectorSubcoreMesh(core_axis_name="c", subcore_axis_name="s")
    f32 = jnp.float32
    return pl.kernel(
        _combine_body,
        out_type=jax.ShapeDtypeStruct((T, D_MODEL), f32),
        mesh=mesh,
        compiler_params=pltpu.CompilerParams(needs_layout_passes=False),
        scratch_types=[
            pltpu.VMEM((CNCH, CCH), jnp.int32),
            pltpu.VMEM((CNCH, CCH), jnp.int32),
            pltpu.VMEM((TOK_W,), f32),
            pltpu.VMEM((TOK_W,), f32),
            pltpu.VMEM((CCH, D_MODEL // 2), jnp.int32),
            pltpu.VMEM((CCH, D_MODEL // 2), jnp.int32),
            pltpu.VMEM((CCH, D_MODEL), f32),
            pltpu.VMEM((CCH, D_MODEL // 2), jnp.int32),
            pltpu.VMEM((CCH, D_MODEL // 2), jnp.int32),
            pltpu.VMEM((CCH, D_MODEL), f32),
            pltpu.SemaphoreType.DMA,
            pltpu.SemaphoreType.DMA,
            pltpu.SemaphoreType.DMA,
            pltpu.SemaphoreType.DMA,
        ],
    )(y, sh, sc0, sc1, w0, w1)


def kernel(hidden_states, gate_w, e_score_correction_bias, up_w, down_w,
           shared_up_w, shared_down_w):
    sb, sc = _gate(hidden_states, gate_w, e_score_correction_bias)
    sd0, sd1, sc0, sc1, w0, w1 = _route(sb, sc)
    sh = _shared(hidden_states, shared_up_w, shared_down_w)
    r3 = lambda a: a.reshape(NW, NCH, CHUNK)
    buf = _dispatch(hidden_states, r3(sd0), r3(sd1))
    y = _experts(buf, up_w, down_w)
    rc = lambda a: a.reshape(NW, CNCH, CCH)
    return _combine(y, sh, rc(sc0), rc(sc1),
                    w0.reshape(NW, TOK_W), w1.reshape(NW, TOK_W))


# speedup vs baseline: 3.2889x; 3.2889x over previous
import jax
import jax.numpy as jnp
from jax import lax
from jax.experimental import pallas as pl
from jax.experimental.pallas import tpu as pltpu
from jax.experimental.pallas import tpu_sc as plsc

E = 64
N_GROUP = 8
GSZ = E // N_GROUP
D_MODEL = 1024
D_FF = 512
T = 4096
CAP = 256
SCALE = 2.0

BUF_ROWS = E * CAP + CAP
DUMMY = E * CAP

NW = 32
TOK_W = T // NW
CHUNK = 32
NCH = TOK_W // CHUNK

RT = 16
RTOK = T // RT
RT1 = 32
R1TOK = T // RT1
NEGINF = float("-inf")


def _gate_body(x_ref, gw_ref, b_ref, sb_ref, sc_ref):
    x = x_ref[...]
    lg = lax.dot_general(gw_ref[...], x, (((0,), (1,)), ((), ())))
    sg = 1.0 / (1.0 + jnp.exp(-lg))
    sc_ref[...] = sg
    sb_ref[...] = sg + b_ref[...]


def _gate(x, gate_w, bias):
    TB = 1024
    return pl.pallas_call(
        _gate_body,
        grid=(T // TB,),
        in_specs=[
            pl.BlockSpec((TB, D_MODEL), lambda i: (i, 0)),
            pl.BlockSpec((D_MODEL, E), lambda i: (0, 0)),
            pl.BlockSpec((E, 1), lambda i: (0, 0)),
        ],
        out_specs=[
            pl.BlockSpec((E, TB), lambda i: (0, i)),
            pl.BlockSpec((E, TB), lambda i: (0, i)),
        ],
        out_shape=[
            jax.ShapeDtypeStruct((E, T), jnp.float32),
            jax.ShapeDtypeStruct((E, T), jnp.float32),
        ],
    )(x, gate_w, bias.reshape(E, 1))


def _shared_body(x_ref, su_ref, sd_ref, sh_ref):
    x = x_ref[...]
    h = jnp.dot(x, su_ref[...])
    h = jnp.square(jnp.maximum(h, 0.0))
    sh_ref[...] = jnp.dot(h, sd_ref[...])


def _shared(x, su, sd):
    TB = 512
    return pl.pallas_call(
        _shared_body,
        grid=(T // TB,),
        in_specs=[
            pl.BlockSpec((TB, D_MODEL), lambda i: (i, 0)),
            pl.BlockSpec((D_MODEL, D_MODEL), lambda i: (0, 0)),
            pl.BlockSpec((D_MODEL, D_MODEL), lambda i: (0, 0)),
        ],
        out_specs=pl.BlockSpec((TB, D_MODEL), lambda i: (i, 0)),
        out_shape=jax.ShapeDtypeStruct((T, D_MODEL), jnp.float32),
    )(x, su, sd)


def _route1_body(sb_hbm, sc_hbm,
                 ids_hbm, lp_hbm, w0_hbm, w1_hbm, cnt_hbm,
                 sb_v, sw_v,
                 ids_v, lp_v, w0_v, w1_v, cnt_v):
    c0 = lax.axis_index("c")
    s = lax.axis_index("s")
    wid = s * 2 + c0
    tok0 = wid * R1TOK
    lane = lax.iota(jnp.int32, 16)

    pltpu.sync_copy(sb_hbm.at[:, pl.ds(tok0, R1TOK)], sb_v)
    pltpu.sync_copy(sc_hbm.at[:, pl.ds(tok0, R1TOK)], sw_v)

    def batch_body(j, _):
        sl16 = pl.ds(j * 16, 16)
        gs = []
        for g in range(N_GROUP):
            m1 = sb_v[g * GSZ, sl16]
            m2 = jnp.full((16,), NEGINF, jnp.float32)
            for i in range(1, GSZ):
                sv = sb_v[g * GSZ + i, sl16]
                m2 = jnp.maximum(m2, jnp.minimum(m1, sv))
                m1 = jnp.maximum(m1, sv)
            gs.append(m1 + m2)
        t1 = jnp.full((16,), NEGINF, jnp.float32)
        t2 = t1
        t3 = t1
        t4 = t1
        for g in range(N_GROUP):
            xg = gs[g]
            n1 = jnp.maximum(t1, xg)
            r = jnp.minimum(t1, xg)
            n2 = jnp.maximum(t2, r)
            r = jnp.minimum(t2, r)
            n3 = jnp.maximum(t3, r)
            r = jnp.minimum(t3, r)
            t4 = jnp.maximum(t4, r)
            t1, t2, t3 = n1, n2, n3
        gmask = [gs[g] >= t4 for g in range(N_GROUP)]
        m1 = jnp.full((16,), NEGINF, jnp.float32)
        m2 = m1
        i1 = jnp.zeros((16,), jnp.int32)
        i2 = i1
        u1 = jnp.zeros((16,), jnp.float32)
        u2 = u1
        for e in range(E):
            sv = jnp.where(gmask[e // GSZ], sb_v[e, sl16], NEGINF)
            wv = sw_v[e, sl16]
            ev = jnp.full((16,), e, jnp.int32)
            gt = sv > m1
            cv = jnp.where(gt, m1, sv)
            ci = jnp.where(gt, i1, ev)
            cw = jnp.where(gt, u1, wv)
            m1 = jnp.where(gt, sv, m1)
            i1 = jnp.where(gt, ev, i1)
            u1 = jnp.where(gt, wv, u1)
            gt2 = cv > m2
            m2 = jnp.where(gt2, cv, m2)
            i2 = jnp.where(gt2, ci, i2)
            u2 = jnp.where(gt2, cw, u2)
        inv = SCALE / (u1 + u2)
        fl = lane * 2 + j * 32
        plsc.store_scatter(ids_v, [fl], i1)
        plsc.store_scatter(ids_v, [fl + 1], i2)
        w0_v[sl16] = u1 * inv
        w1_v[sl16] = u2 * inv
        return 0

    lax.fori_loop(0, R1TOK // 16, batch_body, 0)

    for c in range(E // 16):
        cnt_v[pl.ds(c * 16, 16)] = jnp.zeros((16,), jnp.int32)

    def pos_body(m, _):
        sl16 = pl.ds(m * 16, 16)
        ev = ids_v[sl16]
        c, last = plsc.scan_count(ev)
        basec = plsc.load_gather(cnt_v, [ev])
        lp_v[sl16] = basec + c - 1
        plsc.store_scatter(cnt_v, [ev], basec + c, mask=last)
        return 0

    lax.fori_loop(0, 2 * R1TOK // 16, pos_body, 0)

    asl = pl.ds(2 * tok0, 2 * R1TOK)
    tsl = pl.ds(tok0, R1TOK)
    pltpu.sync_copy(ids_v, ids_hbm.at[asl])
    pltpu.sync_copy(lp_v, lp_hbm.at[asl])
    pltpu.sync_copy(w0_v, w0_hbm.at[tsl])
    pltpu.sync_copy(w1_v, w1_hbm.at[tsl])
    pltpu.sync_copy(cnt_v, cnt_hbm.at[wid])


def _route2_body(ids_hbm, lp_hbm, w0_hbm, w1_hbm, cnt_hbm,
                 sd0_hbm, sd1_hbm, sc0_hbm, sc1_hbm, w0o_hbm, w1o_hbm,
                 ids_v, lp_v, w0_v, w1_v, all_v, base_v, base2_v,
                 sl_d0, sl_d1, sl_c0, sl_c1):
    s = lax.axis_index("s")
    tok0 = s * RTOK
    lane = lax.iota(jnp.int32, 16)

    asl = pl.ds(2 * tok0, 2 * RTOK)
    tsl = pl.ds(tok0, RTOK)
    pltpu.sync_copy(ids_hbm.at[asl], ids_v)
    pltpu.sync_copy(lp_hbm.at[asl], lp_v)
    pltpu.sync_copy(w0_hbm.at[tsl], w0_v)
    pltpu.sync_copy(w1_hbm.at[tsl], w1_v)
    pltpu.sync_copy(cnt_hbm, all_v)

    for c in range(E // 16):
        acc = jnp.zeros((16,), jnp.int32)
        for r in range(RT1):
            pred = jnp.where(jnp.int32(r) < 2 * s, 1, 0).astype(jnp.int32)
            acc = acc + all_v[r, pl.ds(c * 16, 16)] * pred
        base_v[pl.ds(c * 16, 16)] = acc
        half = all_v[2 * s, pl.ds(c * 16, 16)]
        base2_v[pl.ds(c * 16, 16)] = acc + half

    for j in range(RTOK // 16):
        sl16 = pl.ds(j * 16, 16)
        bref = base_v if j < (RTOK // 32) else base2_v
        for k, (w_v, sd_v, sc_slot_v) in enumerate(
            ((w0_v, sl_d0, sl_c0), (w1_v, sl_d1, sl_c1))
        ):
            fl = lane * 2 + (j * 32 + k)
            ev = plsc.load_gather(ids_v, [fl])
            g = plsc.load_gather(bref, [ev]) + plsc.load_gather(lp_v, [fl])
            valid = g < CAP
            posc = jnp.minimum(g, CAP - 1)
            slotc = ev * CAP + posc
            sd_v[sl16] = jnp.where(valid, slotc, DUMMY)
            sc_slot_v[sl16] = slotc
            w_v[sl16] = jnp.where(valid, w_v[sl16], 0.0)

    pltpu.sync_copy(sl_d0, sd0_hbm.at[tsl])
    pltpu.sync_copy(sl_d1, sd1_hbm.at[tsl])
    pltpu.sync_copy(sl_c0, sc0_hbm.at[tsl])
    pltpu.sync_copy(sl_c1, sc1_hbm.at[tsl])
    pltpu.sync_copy(w0_v, w0o_hbm.at[tsl])
    pltpu.sync_copy(w1_v, w1o_hbm.at[tsl])


def _route(sb, sc):
    mesh = plsc.VectorSubcoreMesh(core_axis_name="c", subcore_axis_name="s")
    i32 = jnp.int32
    f32 = jnp.float32
    cp = pltpu.CompilerParams(needs_layout_passes=False)
    ids, lp, w0, w1, cnts = pl.kernel(
        _route1_body,
        out_type=[
            jax.ShapeDtypeStruct((2 * T,), i32),
            jax.ShapeDtypeStruct((2 * T,), i32),
            jax.ShapeDtypeStruct((T,), f32),
            jax.ShapeDtypeStruct((T,), f32),
            jax.ShapeDtypeStruct((RT1, E), i32),
        ],
        mesh=mesh,
        compiler_params=cp,
        scratch_types=[
            pltpu.VMEM((E, R1TOK), f32),
            pltpu.VMEM((E, R1TOK), f32),
            pltpu.VMEM((2 * R1TOK,), i32),
            pltpu.VMEM((2 * R1TOK,), i32),
            pltpu.VMEM((R1TOK,), f32),
            pltpu.VMEM((R1TOK,), f32),
            pltpu.VMEM((E,), i32),
        ],
    )(sb, sc)
    return pl.kernel(
        _route2_body,
        out_type=[jax.ShapeDtypeStruct((T,), i32)] * 4
        + [jax.ShapeDtypeStruct((T,), f32)] * 2,
        mesh=plsc.VectorSubcoreMesh(
            core_axis_name="c", subcore_axis_name="s", num_cores=1),
        compiler_params=cp,
        scratch_types=[
            pltpu.VMEM((2 * RTOK,), i32),
            pltpu.VMEM((2 * RTOK,), i32),
            pltpu.VMEM((RTOK,), f32),
            pltpu.VMEM((RTOK,), f32),
            pltpu.VMEM((RT1, E), i32),
            pltpu.VMEM((E,), i32),
            pltpu.VMEM((E,), i32),
            pltpu.VMEM((RTOK,), i32),
            pltpu.VMEM((RTOK,), i32),
            pltpu.VMEM((RTOK,), i32),
            pltpu.VMEM((RTOK,), i32),
        ],
    )(ids, lp, w0, w1, cnts)


def _dispatch_body(x_hbm, sd0_hbm, sd1_hbm, buf_hbm, idx0_v, idx1_v,
                   rows0, rows1, semL0, semL1, semS0, semS1):
    c = lax.axis_index("c")
    s = lax.axis_index("s")
    wid = s * 2 + c
    pltpu.sync_copy(sd0_hbm.at[wid], idx0_v)
    pltpu.sync_copy(sd1_hbm.at[wid], idx1_v)
    rows = (rows0, rows1)
    semL = (semL0, semL1)
    semS = (semS0, semS1)
    lds = {}
    scats = {}
    lds[0] = pltpu.async_copy(
        x_hbm.at[pl.ds(wid * TOK_W, CHUNK)], rows0, semL0)
    for jj in range(NCH):
        b = jj & 1
        if jj + 1 < NCH:
            if jj >= 1:
                scats[jj - 1][0].wait()
                scats[jj - 1][1].wait()
            lds[jj + 1] = pltpu.async_copy(
                x_hbm.at[pl.ds(wid * TOK_W + (jj + 1) * CHUNK, CHUNK)],
                rows[1 - b], semL[1 - b])
        lds[jj].wait()
        scats[jj] = (
            pltpu.async_copy(rows[b], buf_hbm.at[idx0_v.at[jj]], semS[b]),
            pltpu.async_copy(rows[b], buf_hbm.at[idx1_v.at[jj]], semS[b]),
        )
    for jj in (NCH - 2, NCH - 1):
        scats[jj][0].wait()
        scats[jj][1].wait()


def _dispatch(x, sd0, sd1):
    mesh = plsc.VectorSubcoreMesh(core_axis_name="c", subcore_axis_name="s")
    return pl.kernel(
        _dispatch_body,
        out_type=jax.ShapeDtypeStruct((BUF_ROWS, D_MODEL), jnp.float32),
        mesh=mesh,
        scratch_types=[
            pltpu.VMEM((NCH, CHUNK), jnp.int32),
            pltpu.VMEM((NCH, CHUNK), jnp.int32),
            pltpu.VMEM((CHUNK, D_MODEL), jnp.float32),
            pltpu.VMEM((CHUNK, D_MODEL), jnp.float32),
            pltpu.SemaphoreType.DMA,
            pltpu.SemaphoreType.DMA,
            pltpu.SemaphoreType.DMA,
            pltpu.SemaphoreType.DMA,
        ],
    )(x, sd0, sd1)


def _experts_body(buf_ref, up_ref, dn_ref, y_ref):
    h = jnp.dot(buf_ref[...], up_ref[0])
    h = jnp.square(jnp.maximum(h, 0.0))
    y_ref[...] = jnp.dot(h, dn_ref[0])


def _experts(buf, up_w, down_w):
    return pl.pallas_call(
        _experts_body,
        grid=(E,),
        in_specs=[
            pl.BlockSpec((CAP, D_MODEL), lambda i: (i, 0)),
            pl.BlockSpec((1, D_MODEL, D_FF), lambda i: (i, 0, 0)),
            pl.BlockSpec((1, D_FF, D_MODEL), lambda i: (i, 0, 0)),
        ],
        out_specs=pl.BlockSpec((CAP, D_MODEL), lambda i: (i, 0)),
        out_shape=jax.ShapeDtypeStruct((E * CAP, D_MODEL), jnp.float32),
    )(buf, up_w, down_w)


CCH = 16
CNCH = TOK_W // CCH


def _combine_body(y_hbm, sh_hbm, sc0_hbm, sc1_hbm, w0_hbm, w1_hbm, out_hbm,
                  idx0_v, idx1_v, w0_v, w1_v,
                  ya0, yb0, acc0, ya1, yb1, acc1,
                  semG0, semG1, semO0, semO1):
    c = lax.axis_index("c")
    s = lax.axis_index("s")
    wid = s * 2 + c
    pltpu.sync_copy(sc0_hbm.at[wid], idx0_v)
    pltpu.sync_copy(sc1_hbm.at[wid], idx1_v)
    pltpu.sync_copy(w0_hbm.at[wid], w0_v)
    pltpu.sync_copy(w1_hbm.at[wid], w1_v)
    ya = (ya0, ya1)
    yb = (yb0, yb1)
    acc = (acc0, acc1)
    semG = (semG0, semG1)
    semO = (semO0, semO1)

    def issue(jj, b):
        base = wid * TOK_W + jj * CCH
        return (
            pltpu.async_copy(y_hbm.at[idx0_v.at[jj]], ya[b], semG[b]),
            pltpu.async_copy(y_hbm.at[idx1_v.at[jj]], yb[b], semG[b]),
            pltpu.async_copy(sh_hbm.at[pl.ds(base, CCH)], acc[b], semG[b]),
        )

    gaths = {0: issue(0, 0)}
    stores = {}
    for jj in range(CNCH):
        b = jj & 1
        if jj + 1 < CNCH:
            if jj >= 1:
                stores[jj - 1].wait()
            gaths[jj + 1] = issue(jj + 1, 1 - b)
        for d in gaths[jj]:
            d.wait()

        def row_body(t, _):
            fl = jnp.full((16,), jj * CCH + t, jnp.int32)
            w0s = plsc.load_gather(w0_v, [fl])
            w1s = plsc.load_gather(w1_v, [fl])
            av = acc[b]
            yav = ya[b]
            ybv = yb[b]
            for v in range(D_MODEL // 16):
                sl = pl.ds(v * 16, 16)
                av[t, sl] = av[t, sl] + w0s * yav[t, sl] + w1s * ybv[t, sl]
            return 0

        lax.fori_loop(0, CCH, row_body, 0)
        stores[jj] = pltpu.async_copy(
            acc[b], out_hbm.at[pl.ds(wid * TOK_W + jj * CCH, CCH)], semO[b])
    stores[CNCH - 2].wait()
    stores[CNCH - 1].wait()


def _combine(y, sh, sc0, sc1, w0, w1):
    mesh = plsc.VectorSubcoreMesh(core_axis_name="c", subcore_axis_name="s")
    f32 = jnp.float32
    return pl.kernel(
        _combine_body,
        out_type=jax.ShapeDtypeStruct((T, D_MODEL), f32),
        mesh=mesh,
        compiler_params=pltpu.CompilerParams(needs_layout_passes=False),
        scratch_types=[
            pltpu.VMEM((CNCH, CCH), jnp.int32),
            pltpu.VMEM((CNCH, CCH), jnp.int32),
            pltpu.VMEM((TOK_W,), f32),
            pltpu.VMEM((TOK_W,), f32),
            pltpu.VMEM((CCH, D_MODEL), f32),
            pltpu.VMEM((CCH, D_MODEL), f32),
            pltpu.VMEM((CCH, D_MODEL), f32),
            pltpu.VMEM((CCH, D_MODEL), f32),
            pltpu.VMEM((CCH, D_MODEL), f32),
            pltpu.VMEM((CCH, D_MODEL), f32),
            pltpu.SemaphoreType.DMA,
            pltpu.SemaphoreType.DMA,
            pltpu.SemaphoreType.DMA,
            pltpu.SemaphoreType.DMA,
        ],
    )(y, sh, sc0, sc1, w0, w1)


def kernel(hidden_states, gate_w, e_score_correction_bias, up_w, down_w,
           shared_up_w, shared_down_w):
    sb, sc = _gate(hidden_states, gate_w, e_score_correction_bias)
    sd0, sd1, sc0, sc1, w0, w1 = _route(sb, sc)
    sh = _shared(hidden_states, shared_up_w, shared_down_w)
    r3 = lambda a: a.reshape(NW, NCH, CHUNK)
    buf = _dispatch(hidden_states, r3(sd0), r3(sd1))
    y = _experts(buf, up_w, down_w)
    rc = lambda a: a.reshape(NW, CNCH, CCH)
    return _combine(y, sh, rc(sc0), rc(sc1),
                    w0.reshape(NW, TOK_W), w1.reshape(NW, TOK_W))
